# baseline (speedup 1.0000x reference)
import jax
import jax.numpy as jnp
from jax.experimental import pallas as pl
from jax.experimental.pallas import tpu as pltpu

N = 10000
BM = 128
NBLK = -(-N // BM)
NPAD = NBLK * BM
CACHE_SLABS = 16
CACHE_ROWS = CACHE_SLABS * BM
NSTREAM = NBLK - CACHE_SLABS

_CONTRACT_LAST = (((1,), (1,)), ((), ()))
_CONTRACT_00 = (((0,), (0,)), ((), ()))


def _gcn(adj_ref, x_ref, W1_ref, b1c_ref, W2_ref, b2r_ref, out_ref,
         sT_ref, gT_ref, cache_ref, outcT_ref, gT16_ref):
    p = pl.program_id(0)
    i = pl.program_id(1)

    @pl.when((p == 0) & (i == 0))
    def _():
        s = jnp.dot(x_ref[:], W1_ref[:], preferred_element_type=jnp.float32)
        sT_ref[:] = s.T

    @pl.when(p == 0)
    def _():
        hT = jax.lax.dot_general(sT_ref[:], adj_ref[:], _CONTRACT_LAST,
                                 preferred_element_type=jnp.float32)
        hT = jnp.maximum(hT + b1c_ref[:], 0.0)
        gT_ref[:, pl.ds(i * BM, BM)] = jax.lax.dot_general(
            W2_ref[:], hT, _CONTRACT_00, preferred_element_type=jnp.float32)

        @pl.when(i < CACHE_SLABS)
        def _():
            cache_ref[pl.ds(i * BM, BM), :] = adj_ref[:].astype(jnp.bfloat16)

    @pl.when(p == 1)
    def _():
        @pl.when(i < NSTREAM)
        def _():
            oT = jax.lax.dot_general(gT_ref[:, :N], adj_ref[:],
                                     _CONTRACT_LAST,
                                     preferred_element_type=jnp.float32)
            out_ref[:] = oT.T + b2r_ref[:]

        @pl.when(i == 0)
        def _():
            gT16_ref[:] = gT_ref[:, :N].astype(jnp.bfloat16)

        @pl.when(i < CACHE_SLABS)
        def _():
            outcT_ref[:, pl.ds(i * BM, BM)] = jax.lax.dot_general(
                gT16_ref[:], cache_ref[pl.ds(i * BM, BM), :], _CONTRACT_LAST,
                preferred_element_type=jnp.float32)

        @pl.when(i >= NSTREAM)
        def _():
            j = i - NSTREAM
            out_ref[:] = outcT_ref[:, pl.ds(j * BM, BM)].T + b2r_ref[:]


def kernel(x, adj, W1, b1, W2, b2):
    nfeat = x.shape[1]
    nhid = W1.shape[1]
    nclass = W2.shape[1]
    b1c = b1.reshape(nhid, 1)
    b2r = b2.reshape(1, nclass)

    full = lambda shape: pl.BlockSpec(shape, lambda p, i: (0, 0))

    def adj_idx(p, i):
        return (jnp.where(p == 0, i,
                          jnp.minimum(i + CACHE_SLABS, NBLK - 1)), 0)

    def out_idx(p, i):
        return (jnp.where(p == 0, CACHE_SLABS,
                          jnp.where(i < NSTREAM, i + CACHE_SLABS,
                                    i - NSTREAM)), 0)

    out = pl.pallas_call(
        _gcn,
        grid=(2, NBLK),
        in_specs=[
            pl.BlockSpec((BM, N), adj_idx),
            full((N, nfeat)),
            full((nfeat, nhid)),
            full((nhid, 1)),
            full((nhid, nclass)),
            full((1, nclass)),
        ],
        out_specs=pl.BlockSpec((BM, nclass), out_idx),
        out_shape=jax.ShapeDtypeStruct((N, nclass), jnp.float32),
        scratch_shapes=[
            pltpu.VMEM((nhid, N), jnp.float32),
            pltpu.VMEM((nclass, NPAD), jnp.float32),
            pltpu.VMEM((CACHE_ROWS, N), jnp.bfloat16),
            pltpu.VMEM((nclass, CACHE_ROWS), jnp.float32),
            pltpu.VMEM((nclass, N), jnp.bfloat16),
        ],
        compiler_params=pltpu.CompilerParams(
            dimension_semantics=("arbitrary", "arbitrary")),
    )(adj, x, W1, b1c, W2, b2r)

    return out


# speedup vs baseline: 1.1382x; 1.1382x over previous
import jax
import jax.numpy as jnp
from jax.experimental import pallas as pl
from jax.experimental.pallas import tpu as pltpu

N = 10000
BM = 512
NBLK = -(-N // BM)
NPAD = NBLK * BM
CACHE_SLABS = 1
CACHE_ROWS = CACHE_SLABS * BM
NSTREAM = NBLK - CACHE_SLABS

_CONTRACT_LAST = (((1,), (1,)), ((), ()))
_CONTRACT_00 = (((0,), (0,)), ((), ()))


def _gcn(adj_ref, x_ref, W1_ref, b1c_ref, W2_ref, b2r_ref, out_ref,
         sT_ref, gT_ref, cache_ref, outcT_ref, gT16_ref):
    p = pl.program_id(0)
    i = pl.program_id(1)

    @pl.when((p == 0) & (i == 0))
    def _():
        s = jnp.dot(x_ref[:], W1_ref[:], preferred_element_type=jnp.float32)
        sT_ref[:] = s.T

    @pl.when(p == 0)
    def _():
        hT = jax.lax.dot_general(sT_ref[:], adj_ref[:], _CONTRACT_LAST,
                                 preferred_element_type=jnp.float32)
        hT = jnp.maximum(hT + b1c_ref[:], 0.0)
        gT_ref[:, pl.ds(i * BM, BM)] = jax.lax.dot_general(
            W2_ref[:], hT, _CONTRACT_00, preferred_element_type=jnp.float32)

        @pl.when(i < CACHE_SLABS)
        def _():
            cache_ref[pl.ds(i * BM, BM), :] = adj_ref[:].astype(jnp.bfloat16)

    @pl.when(p == 1)
    def _():
        @pl.when(i < NSTREAM)
        def _():
            oT = jax.lax.dot_general(gT_ref[:, :N], adj_ref[:],
                                     _CONTRACT_LAST,
                                     preferred_element_type=jnp.float32)
            out_ref[:] = oT.T + b2r_ref[:]

        @pl.when(i == 0)
        def _():
            gT16_ref[:] = gT_ref[:, :N].astype(jnp.bfloat16)

        @pl.when(i < CACHE_SLABS)
        def _():
            outcT_ref[:, pl.ds(i * BM, BM)] = jax.lax.dot_general(
                gT16_ref[:], cache_ref[pl.ds(i * BM, BM), :], _CONTRACT_LAST,
                preferred_element_type=jnp.float32)

        @pl.when(i >= NSTREAM)
        def _():
            j = i - NSTREAM
            out_ref[:] = outcT_ref[:, pl.ds(j * BM, BM)].T + b2r_ref[:]


def kernel(x, adj, W1, b1, W2, b2):
    nfeat = x.shape[1]
    nhid = W1.shape[1]
    nclass = W2.shape[1]
    b1c = b1.reshape(nhid, 1)
    b2r = b2.reshape(1, nclass)

    full = lambda shape: pl.BlockSpec(shape, lambda p, i: (0, 0))

    def adj_idx(p, i):
        return (jnp.where(p == 0, i,
                          jnp.minimum(i + CACHE_SLABS, NBLK - 1)), 0)

    def out_idx(p, i):
        return (jnp.where(p == 0, CACHE_SLABS,
                          jnp.where(i < NSTREAM, i + CACHE_SLABS,
                                    i - NSTREAM)), 0)

    out = pl.pallas_call(
        _gcn,
        grid=(2, NBLK),
        in_specs=[
            pl.BlockSpec((BM, N), adj_idx),
            full((N, nfeat)),
            full((nfeat, nhid)),
            full((nhid, 1)),
            full((nhid, nclass)),
            full((1, nclass)),
        ],
        out_specs=pl.BlockSpec((BM, nclass), out_idx),
        out_shape=jax.ShapeDtypeStruct((N, nclass), jnp.float32),
        scratch_shapes=[
            pltpu.VMEM((nhid, N), jnp.float32),
            pltpu.VMEM((nclass, NPAD), jnp.float32),
            pltpu.VMEM((CACHE_ROWS, N), jnp.bfloat16),
            pltpu.VMEM((nclass, CACHE_ROWS), jnp.float32),
            pltpu.VMEM((nclass, N), jnp.bfloat16),
        ],
        compiler_params=pltpu.CompilerParams(
            dimension_semantics=("arbitrary", "arbitrary")),
    )(adj, x, W1, b1c, W2, b2r)

    return out


# speedup vs baseline: 1.1449x; 1.0059x over previous
import jax
import jax.numpy as jnp
from jax.experimental import pallas as pl
from jax.experimental.pallas import tpu as pltpu

N = 10000
BM = 256
NBLK = -(-N // BM)
NPAD = NBLK * BM
XBM = 1280
XCH = -(-N // XBM)
CACHE_SLABS = 7
CACHE_ROWS = CACHE_SLABS * BM
NSTREAM = NBLK - CACHE_SLABS

S0 = XCH
O0 = S0 + NBLK
FL0 = O0 + NSTREAM
STEPS = O0 + NBLK

_CONTRACT_LAST = (((1,), (1,)), ((), ()))
_CONTRACT_00 = (((0,), (0,)), ((), ()))


def _gcn(adj_ref, x_ref, W1_ref, b1c_ref, W2_ref, b2r_ref, out_ref,
         sT_ref, gT_ref, cache_ref, outcT_ref, gT16_ref):
    i = pl.program_id(0)

    @pl.when(i < S0)
    def _():
        s = jnp.dot(x_ref[:], W1_ref[:], preferred_element_type=jnp.float32)
        sT_ref[:, pl.ds(i * XBM, XBM)] = s.T

    @pl.when((i >= S0) & (i < O0))
    def _():
        ih = i - S0
        hT = jax.lax.dot_general(sT_ref[:, :N], adj_ref[:], _CONTRACT_LAST,
                                 preferred_element_type=jnp.float32)
        hT = jnp.maximum(hT + b1c_ref[:], 0.0)
        gT_ref[:, pl.ds(ih * BM, BM)] = jax.lax.dot_general(
            W2_ref[:], hT, _CONTRACT_00, preferred_element_type=jnp.float32)

        @pl.when(ih < CACHE_SLABS)
        def _():
            cache_ref[pl.ds(ih * BM, BM), :] = adj_ref[:].astype(jnp.bfloat16)

    @pl.when(i >= O0)
    def _():
        io = i - O0

        @pl.when(io < NSTREAM)
        def _():
            oT = jax.lax.dot_general(gT_ref[:, :N], adj_ref[:],
                                     _CONTRACT_LAST,
                                     preferred_element_type=jnp.float32)
            out_ref[:] = oT.T + b2r_ref[:]

        @pl.when(io == 0)
        def _():
            gT16_ref[:] = gT_ref[:, :N].astype(jnp.bfloat16)

        @pl.when(io < CACHE_SLABS)
        def _():
            outcT_ref[:, pl.ds(io * BM, BM)] = jax.lax.dot_general(
                gT16_ref[:], cache_ref[pl.ds(io * BM, BM), :],
                _CONTRACT_LAST, preferred_element_type=jnp.float32)

        @pl.when(io >= NSTREAM)
        def _():
            j = io - NSTREAM
            out_ref[:] = outcT_ref[:, pl.ds(j * BM, BM)].T + b2r_ref[:]


def kernel(x, adj, W1, b1, W2, b2):
    nfeat = x.shape[1]
    nhid = W1.shape[1]
    nclass = W2.shape[1]
    b1c = b1.reshape(nhid, 1)
    b2r = b2.reshape(1, nclass)

    full = lambda shape: pl.BlockSpec(shape, lambda i: (0, 0))

    def adj_idx(i):
        return (jnp.where(i < S0, 0,
                          jnp.where(i < O0, i - S0,
                                    jnp.minimum(i - O0 + CACHE_SLABS,
                                                NBLK - 1))), 0)

    def x_idx(i):
        return (jnp.minimum(i, XCH - 1), 0)

    def out_idx(i):
        return (jnp.where(i < O0, CACHE_SLABS,
                          jnp.where(i < FL0, i - O0 + CACHE_SLABS,
                                    i - FL0)), 0)

    out = pl.pallas_call(
        _gcn,
        grid=(STEPS,),
        in_specs=[
            pl.BlockSpec((BM, N), adj_idx),
            pl.BlockSpec((XBM, nfeat), x_idx),
            full((nfeat, nhid)),
            full((nhid, 1)),
            full((nhid, nclass)),
            full((1, nclass)),
        ],
        out_specs=pl.BlockSpec((BM, nclass), out_idx),
        out_shape=jax.ShapeDtypeStruct((N, nclass), jnp.float32),
        scratch_shapes=[
            pltpu.VMEM((nhid, NPAD), jnp.float32),
            pltpu.VMEM((nclass, NPAD), jnp.float32),
            pltpu.VMEM((CACHE_ROWS, N), jnp.bfloat16),
            pltpu.VMEM((nclass, CACHE_ROWS), jnp.float32),
            pltpu.VMEM((nclass, N), jnp.bfloat16),
        ],
        compiler_params=pltpu.CompilerParams(
            dimension_semantics=("arbitrary",)),
    )(adj, x, W1, b1c, W2, b2r)

    return out


# speedup vs baseline: 1.1537x; 1.0076x over previous
import jax
import jax.numpy as jnp
from jax.experimental import pallas as pl
from jax.experimental.pallas import tpu as pltpu

N = 10000
BM = 256
NBLK = -(-N // BM)
NPAD = NBLK * BM
CACHE_SLABS = 6
CACHE_ROWS = CACHE_SLABS * BM
NSTREAM = NBLK - CACHE_SLABS

_CONTRACT_LAST = (((1,), (1,)), ((), ()))
_CONTRACT_00 = (((0,), (0,)), ((), ()))


def _gcn(adj_ref, x_ref, W1_ref, b1c_ref, W2_ref, b2r_ref, out_ref,
         sT_ref, gT_ref, cache_ref, outcT_ref, gT16_ref):
    p = pl.program_id(0)
    i = pl.program_id(1)

    @pl.when((p == 0) & (i == 0))
    def _():
        s = jnp.dot(x_ref[:], W1_ref[:], preferred_element_type=jnp.float32)
        sT_ref[:] = s.T

    @pl.when(p == 0)
    def _():
        hT = jax.lax.dot_general(sT_ref[:], adj_ref[:], _CONTRACT_LAST,
                                 preferred_element_type=jnp.float32)
        hT = jnp.maximum(hT + b1c_ref[:], 0.0)
        gT_ref[:, pl.ds(i * BM, BM)] = jax.lax.dot_general(
            W2_ref[:], hT, _CONTRACT_00, preferred_element_type=jnp.float32)

        @pl.when(i < CACHE_SLABS)
        def _():
            cache_ref[pl.ds(i * BM, BM), :] = adj_ref[:].astype(jnp.bfloat16)

    @pl.when(p == 1)
    def _():
        @pl.when(i < NSTREAM)
        def _():
            oT = jax.lax.dot_general(gT_ref[:, :N], adj_ref[:],
                                     _CONTRACT_LAST,
                                     preferred_element_type=jnp.float32)
            out_ref[:] = oT.T + b2r_ref[:]

        @pl.when(i == 0)
        def _():
            gT16_ref[:] = gT_ref[:, :N].astype(jnp.bfloat16)

        @pl.when(i < CACHE_SLABS)
        def _():
            outcT_ref[:, pl.ds(i * BM, BM)] = jax.lax.dot_general(
                gT16_ref[:], cache_ref[pl.ds(i * BM, BM), :], _CONTRACT_LAST,
                preferred_element_type=jnp.float32)

        @pl.when(i >= NSTREAM)
        def _():
            j = i - NSTREAM
            out_ref[:] = outcT_ref[:, pl.ds(j * BM, BM)].T + b2r_ref[:]


def kernel(x, adj, W1, b1, W2, b2):
    nfeat = x.shape[1]
    nhid = W1.shape[1]
    nclass = W2.shape[1]
    b1c = b1.reshape(nhid, 1)
    b2r = b2.reshape(1, nclass)

    full = lambda shape: pl.BlockSpec(shape, lambda p, i: (0, 0))

    def adj_idx(p, i):
        return (jnp.where(p == 0, i,
                          jnp.minimum(i + CACHE_SLABS, NBLK - 1)), 0)

    def out_idx(p, i):
        return (jnp.where(p == 0, CACHE_SLABS,
                          jnp.where(i < NSTREAM, i + CACHE_SLABS,
                                    i - NSTREAM)), 0)

    out = pl.pallas_call(
        _gcn,
        grid=(2, NBLK),
        in_specs=[
            pl.BlockSpec((BM, N), adj_idx),
            full((N, nfeat)),
            full((nfeat, nhid)),
            full((nhid, 1)),
            full((nhid, nclass)),
            full((1, nclass)),
        ],
        out_specs=pl.BlockSpec((BM, nclass), out_idx),
        out_shape=jax.ShapeDtypeStruct((N, nclass), jnp.float32),
        scratch_shapes=[
            pltpu.VMEM((nhid, N), jnp.float32),
            pltpu.VMEM((nclass, NPAD), jnp.float32),
            pltpu.VMEM((CACHE_ROWS, N), jnp.bfloat16),
            pltpu.VMEM((nclass, CACHE_ROWS), jnp.float32),
            pltpu.VMEM((nclass, N), jnp.bfloat16),
        ],
        compiler_params=pltpu.CompilerParams(
            dimension_semantics=("arbitrary", "arbitrary")),
    )(adj, x, W1, b1c, W2, b2r)

    return out


# speedup vs baseline: 1.1537x; 1.0000x over previous
import jax
import jax.numpy as jnp
from jax.experimental import pallas as pl
from jax.experimental.pallas import tpu as pltpu

N = 10000
BM = 256
NBLK = -(-N // BM)
NPAD = NBLK * BM
CACHE_SLABS = 6
CACHE_ROWS = CACHE_SLABS * BM
NSTREAM = NBLK - CACHE_SLABS

_CONTRACT_LAST = (((1,), (1,)), ((), ()))
_CONTRACT_00 = (((0,), (0,)), ((), ()))


def _gcn(adj_ref, x_ref, W1_ref, b1c_ref, W2_ref, b2r_ref, out_ref,
         sT_ref, gT_ref, cache_ref, outcT_ref, gT16_ref):
    p = pl.program_id(0)
    i = pl.program_id(1)

    @pl.when((p == 0) & (i == 0))
    def _():
        s = jnp.dot(x_ref[:], W1_ref[:], preferred_element_type=jnp.float32)
        sT_ref[:] = s.T

    @pl.when(p == 0)
    def _():
        hT = jax.lax.dot_general(sT_ref[:], adj_ref[:], _CONTRACT_LAST,
                                 preferred_element_type=jnp.float32)
        hT = jnp.maximum(hT + b1c_ref[:], 0.0)
        gT_ref[:, pl.ds(i * BM, BM)] = jax.lax.dot_general(
            W2_ref[:], hT, _CONTRACT_00, preferred_element_type=jnp.float32)

        @pl.when(i < CACHE_SLABS)
        def _():
            cache_ref[pl.ds(i * BM, BM), :] = adj_ref[:].astype(jnp.bfloat16)

    @pl.when(p == 1)
    def _():
        @pl.when(i < NSTREAM)
        def _():
            oT = jax.lax.dot_general(gT_ref[:, :N], adj_ref[:],
                                     _CONTRACT_LAST,
                                     preferred_element_type=jnp.float32)
            out_ref[:] = oT.T + b2r_ref[:]

        @pl.when(i == 0)
        def _():
            gT16_ref[:] = gT_ref[:, :N].astype(jnp.bfloat16)

        @pl.when(i < CACHE_SLABS)
        def _():
            outcT_ref[:, pl.ds(i * BM, BM)] = jax.lax.dot_general(
                gT16_ref[:], cache_ref[pl.ds(i * BM, BM), :], _CONTRACT_LAST,
                preferred_element_type=jnp.float32)

        @pl.when(i >= NSTREAM)
        def _():
            j = i - NSTREAM
            out_ref[:] = outcT_ref[:, pl.ds(j * BM, BM)].T + b2r_ref[:]


def kernel(x, adj, W1, b1, W2, b2):
    nfeat = x.shape[1]
    nhid = W1.shape[1]
    nclass = W2.shape[1]
    b1c = b1.reshape(nhid, 1)
    b2r = b2.reshape(1, nclass)

    full = lambda shape: pl.BlockSpec(shape, lambda p, i: (0, 0))

    def adj_idx(p, i):
        return (jnp.where(p == 0, i,
                          jnp.maximum(NBLK - 1 - i, CACHE_SLABS)), 0)

    def out_idx(p, i):
        return (jnp.where(p == 0, NBLK - 1,
                          jnp.where(i < NSTREAM, NBLK - 1 - i,
                                    i - NSTREAM)), 0)

    out = pl.pallas_call(
        _gcn,
        grid=(2, NBLK),
        in_specs=[
            pl.BlockSpec((BM, N), adj_idx),
            full((N, nfeat)),
            full((nfeat, nhid)),
            full((nhid, 1)),
            full((nhid, nclass)),
            full((1, nclass)),
        ],
        out_specs=pl.BlockSpec((BM, nclass), out_idx),
        out_shape=jax.ShapeDtypeStruct((N, nclass), jnp.float32),
        scratch_shapes=[
            pltpu.VMEM((nhid, N), jnp.float32),
            pltpu.VMEM((nclass, NPAD), jnp.float32),
            pltpu.VMEM((CACHE_ROWS, N), jnp.bfloat16),
            pltpu.VMEM((nclass, CACHE_ROWS), jnp.float32),
            pltpu.VMEM((nclass, N), jnp.bfloat16),
        ],
        compiler_params=pltpu.CompilerParams(
            dimension_semantics=("arbitrary", "arbitrary")),
    )(adj, x, W1, b1c, W2, b2r)

    return out


# speedup vs baseline: 1.1622x; 1.0074x over previous
import jax
import jax.numpy as jnp
from jax.experimental import pallas as pl
from jax.experimental.pallas import tpu as pltpu

N = 10000
BM = 256
NBLK = -(-N // BM)
NPAD = NBLK * BM
CACHE_SLABS = 6
CACHE_ROWS = CACHE_SLABS * BM
NSTREAM = NBLK - CACHE_SLABS

_CONTRACT_LAST = (((1,), (1,)), ((), ()))
_CONTRACT_00 = (((0,), (0,)), ((), ()))


def _gcn(adj_ref, x_ref, W1_ref, b1c_ref, W2_ref, b2r_ref, out_ref,
         sT_ref, gT_ref, cache_ref, outcT_ref, gT16_ref):
    p = pl.program_id(0)
    i = pl.program_id(1)

    @pl.when((p == 0) & (i == 0))
    def _():
        s = jnp.dot(x_ref[:], W1_ref[:], preferred_element_type=jnp.float32)
        sT_ref[:] = s.T

    @pl.when(p == 0)
    def _():
        blk = jnp.where(i == 0, NBLK - 1, i - 1)
        hT = jax.lax.dot_general(sT_ref[:], adj_ref[:], _CONTRACT_LAST,
                                 preferred_element_type=jnp.float32)
        hT = jnp.maximum(hT + b1c_ref[:], 0.0)
        gT_ref[:, pl.ds(blk * BM, BM)] = jax.lax.dot_general(
            W2_ref[:], hT, _CONTRACT_00, preferred_element_type=jnp.float32)

        @pl.when((i >= 1) & (i < CACHE_SLABS + 1))
        def _():
            cache_ref[pl.ds((i - 1) * BM, BM), :] = (
                adj_ref[:].astype(jnp.bfloat16))

    @pl.when(p == 1)
    def _():
        @pl.when(i < NSTREAM)
        def _():
            oT = jax.lax.dot_general(gT_ref[:, :N], adj_ref[:],
                                     _CONTRACT_LAST,
                                     preferred_element_type=jnp.float32)
            out_ref[:] = oT.T + b2r_ref[:]

        @pl.when(i == 0)
        def _():
            gT16_ref[:] = gT_ref[:, :N].astype(jnp.bfloat16)

        @pl.when(i < CACHE_SLABS)
        def _():
            outcT_ref[:, pl.ds(i * BM, BM)] = jax.lax.dot_general(
                gT16_ref[:], cache_ref[pl.ds(i * BM, BM), :], _CONTRACT_LAST,
                preferred_element_type=jnp.float32)

        @pl.when(i >= NSTREAM)
        def _():
            j = i - NSTREAM
            out_ref[:] = outcT_ref[:, pl.ds(j * BM, BM)].T + b2r_ref[:]


def kernel(x, adj, W1, b1, W2, b2):
    nfeat = x.shape[1]
    nhid = W1.shape[1]
    nclass = W2.shape[1]
    b1c = b1.reshape(nhid, 1)
    b2r = b2.reshape(1, nclass)

    full = lambda shape: pl.BlockSpec(shape, lambda p, i: (0, 0))

    def adj_idx(p, i):
        return (jnp.where(p == 0,
                          jnp.where(i == 0, NBLK - 1, i - 1),
                          jnp.maximum(NBLK - 1 - i, CACHE_SLABS)), 0)

    def out_idx(p, i):
        return (jnp.where(p == 0, NBLK - 1,
                          jnp.where(i < NSTREAM, NBLK - 1 - i,
                                    i - NSTREAM)), 0)

    out = pl.pallas_call(
        _gcn,
        grid=(2, NBLK),
        in_specs=[
            pl.BlockSpec((BM, N), adj_idx),
            full((N, nfeat)),
            full((nfeat, nhid)),
            full((nhid, 1)),
            full((nhid, nclass)),
            full((1, nclass)),
        ],
        out_specs=pl.BlockSpec((BM, nclass), out_idx),
        out_shape=jax.ShapeDtypeStruct((N, nclass), jnp.float32),
        scratch_shapes=[
            pltpu.VMEM((nhid, N), jnp.float32),
            pltpu.VMEM((nclass, NPAD), jnp.float32),
            pltpu.VMEM((CACHE_ROWS, N), jnp.bfloat16),
            pltpu.VMEM((nclass, CACHE_ROWS), jnp.float32),
            pltpu.VMEM((nclass, N), jnp.bfloat16),
        ],
        compiler_params=pltpu.CompilerParams(
            dimension_semantics=("arbitrary", "arbitrary")),
    )(adj, x, W1, b1c, W2, b2r)

    return out


# speedup vs baseline: 1.1665x; 1.0037x over previous
import jax
import jax.numpy as jnp
from jax.experimental import pallas as pl
from jax.experimental.pallas import tpu as pltpu

N = 10000
BM = 256
NBLK = -(-N // BM)
NPAD = NBLK * BM
CACHE_SLABS = 6
CACHE_ROWS = CACHE_SLABS * BM
NSTREAM = NBLK - CACHE_SLABS

_CONTRACT_LAST = (((1,), (1,)), ((), ()))
_CONTRACT_00 = (((0,), (0,)), ((), ()))


def _gcn(adj_ref, x_ref, W1_ref, b1c_ref, W2_ref, b2r_ref, out_ref,
         sT_ref, gT_ref, cache_ref, outcT_ref, gT16_ref):
    p = pl.program_id(0)
    i = pl.program_id(1)

    @pl.when((p == 0) & (i == 0))
    def _():
        s = jnp.dot(x_ref[:], W1_ref[:], preferred_element_type=jnp.float32)
        sT_ref[:] = s.T

    @pl.when(p == 0)
    def _():
        blk = jnp.where(i == 0, NBLK - 1, i - 1)
        hT = jax.lax.dot_general(sT_ref[:], adj_ref[:], _CONTRACT_LAST,
                                 preferred_element_type=jnp.float32)
        hT = jnp.maximum(hT + b1c_ref[:], 0.0)
        gT_ref[:, pl.ds(blk * BM, BM)] = jax.lax.dot_general(
            W2_ref[:], hT, _CONTRACT_00, preferred_element_type=jnp.float32)

        @pl.when((i >= 1) & (i < CACHE_SLABS + 1))
        def _():
            cache_ref[pl.ds((i - 1) * BM, BM), :] = (
                adj_ref[:].astype(jnp.bfloat16))

    @pl.when(p == 1)
    def _():
        @pl.when(i < NSTREAM)
        def _():
            slab = NBLK - 1 - i
            oT = jax.lax.dot_general(gT_ref[:, :N], adj_ref[:],
                                     _CONTRACT_LAST,
                                     preferred_element_type=jnp.float32)
            off = jnp.where(slab % 2 == 1, BM, 0)
            out_ref[pl.ds(off, BM), :] = oT.T + b2r_ref[:]

        @pl.when(i == 0)
        def _():
            gT16_ref[:] = gT_ref[:, :N].astype(jnp.bfloat16)

        @pl.when(i < CACHE_SLABS)
        def _():
            outcT_ref[:, pl.ds(i * BM, BM)] = jax.lax.dot_general(
                gT16_ref[:], cache_ref[pl.ds(i * BM, BM), :], _CONTRACT_LAST,
                preferred_element_type=jnp.float32)

        @pl.when((i >= NSTREAM) & (i < NSTREAM + CACHE_SLABS // 2))
        def _():
            j = i - NSTREAM
            out_ref[:] = outcT_ref[:, pl.ds(j * 2 * BM, 2 * BM)].T + b2r_ref[:]


def kernel(x, adj, W1, b1, W2, b2):
    nfeat = x.shape[1]
    nhid = W1.shape[1]
    nclass = W2.shape[1]
    b1c = b1.reshape(nhid, 1)
    b2r = b2.reshape(1, nclass)

    full = lambda shape: pl.BlockSpec(shape, lambda p, i: (0, 0))

    def adj_idx(p, i):
        return (jnp.where(p == 0,
                          jnp.where(i == 0, NBLK - 1, i - 1),
                          jnp.maximum(NBLK - 1 - i, CACHE_SLABS)), 0)

    def out_idx(p, i):
        nob = NBLK // 2
        return (jnp.where(p == 0, nob - 1,
                          jnp.where(i < NSTREAM, nob - 1 - i // 2,
                                    jnp.minimum(i - NSTREAM,
                                                CACHE_SLABS // 2 - 1))), 0)

    out = pl.pallas_call(
        _gcn,
        grid=(2, NBLK),
        in_specs=[
            pl.BlockSpec((BM, N), adj_idx),
            full((N, nfeat)),
            full((nfeat, nhid)),
            full((nhid, 1)),
            full((nhid, nclass)),
            full((1, nclass)),
        ],
        out_specs=pl.BlockSpec((2 * BM, nclass), out_idx),
        out_shape=jax.ShapeDtypeStruct((N, nclass), jnp.float32),
        scratch_shapes=[
            pltpu.VMEM((nhid, N), jnp.float32),
            pltpu.VMEM((nclass, NPAD), jnp.float32),
            pltpu.VMEM((CACHE_ROWS, N), jnp.bfloat16),
            pltpu.VMEM((nclass, CACHE_ROWS), jnp.float32),
            pltpu.VMEM((nclass, N), jnp.bfloat16),
        ],
        compiler_params=pltpu.CompilerParams(
            dimension_semantics=("arbitrary", "arbitrary")),
    )(adj, x, W1, b1c, W2, b2r)

    return out
